# Initial kernel scaffold; baseline (speedup 1.0000x reference)
import jax
import jax.numpy as jnp
from jax.experimental import pallas as pl


def kernel(class_logits, box_regression, proposal_boxes):
    raise NotImplementedError("write your pallas kernel here")


# speedup vs baseline: 21.3969x; 21.3969x over previous
import jax
import jax.numpy as jnp
import numpy as np
from jax.experimental import pallas as pl

N = 5000
NUM_CLASSES = 2
SCORE_THRESH = 0.05
NMS_THRESH = 0.5
DETS = 100
IMG_W, IMG_H = 512.0, 512.0
WX, WY, WW, WH = 10.0, 10.0, 5.0, 5.0
BBOX_XFORM_CLIP = float(np.log(1000.0 / 16.0))

ROWS, LANES = 40, 128
NPAD = ROWS * LANES
NEG_INF = float("-inf")


def _nms_kernel(data_ref, out_ref):
    l0 = data_ref[0]
    l1 = data_ref[1]
    rx = data_ref[2]
    ry = data_ref[3]
    rw = data_ref[4]
    rh = data_ref[5]
    px1 = data_ref[6]
    py1 = data_ref[7]
    px2 = data_ref[8]
    py2 = data_ref[9]

    mx = jnp.maximum(l0, l1)
    e0 = jnp.exp(l0 - mx)
    e1 = jnp.exp(l1 - mx)
    score = e1 / (e0 + e1)

    widths = px2 - px1 + 1.0
    heights = py2 - py1 + 1.0
    ctr_x = px1 + 0.5 * widths
    ctr_y = py1 + 0.5 * heights
    dx = rx / WX
    dy = ry / WY
    dw = jnp.minimum(rw / WW, BBOX_XFORM_CLIP)
    dh = jnp.minimum(rh / WH, BBOX_XFORM_CLIP)
    pcx = dx * widths + ctr_x
    pcy = dy * heights + ctr_y
    pw = jnp.exp(dw) * widths
    ph = jnp.exp(dh) * heights
    x1 = jnp.clip(pcx - 0.5 * pw, 0.0, IMG_W - 1.0)
    y1 = jnp.clip(pcy - 0.5 * ph, 0.0, IMG_H - 1.0)
    x2 = jnp.clip(pcx + 0.5 * pw - 1.0, 0.0, IMG_W - 1.0)
    y2 = jnp.clip(pcy + 0.5 * ph - 1.0, 0.0, IMG_H - 1.0)
    areas = (x2 - x1 + 1.0) * (y2 - y1 + 1.0)

    lin = (jax.lax.broadcasted_iota(jnp.int32, (ROWS, LANES), 0) * LANES
           + jax.lax.broadcasted_iota(jnp.int32, (ROWS, LANES), 1))
    lane = jax.lax.broadcasted_iota(jnp.int32, (1, LANES), 1)

    s0 = jnp.where((score > SCORE_THRESH) & (lin < N), score, NEG_INF)

    def body(i, s):
        m = jnp.max(s)
        valid = m != NEG_INF
        isel = jnp.min(jnp.where(s == m, lin, jnp.int32(NPAD)))
        onehot = lin == isel
        bx1 = jnp.sum(jnp.where(onehot, x1, 0.0))
        by1 = jnp.sum(jnp.where(onehot, y1, 0.0))
        bx2 = jnp.sum(jnp.where(onehot, x2, 0.0))
        by2 = jnp.sum(jnp.where(onehot, y2, 0.0))
        barea = jnp.sum(jnp.where(onehot, areas, 0.0))

        xx1 = jnp.maximum(bx1, x1)
        yy1 = jnp.maximum(by1, y1)
        xx2 = jnp.minimum(bx2, x2)
        yy2 = jnp.minimum(by2, y2)
        w = jnp.maximum(xx2 - xx1 + 1.0, 0.0)
        h = jnp.maximum(yy2 - yy1 + 1.0, 0.0)
        inter = w * h
        iou = inter / (barea + areas - inter)
        s = jnp.where(iou > NMS_THRESH, NEG_INF, s)

        row = jnp.where(lane == 0, bx1,
              jnp.where(lane == 1, by1,
              jnp.where(lane == 2, bx2,
              jnp.where(lane == 3, by2,
              jnp.where(lane == 4, m, 0.0)))))
        out_ref[pl.ds(i, 1), :] = jnp.where(valid, row, 0.0)
        return s

    jax.lax.fori_loop(0, DETS, body, s0, unroll=False)


def _prep(x):
    xt = jnp.transpose(x)
    xt = jnp.pad(xt, ((0, 0), (0, NPAD - N)))
    return xt.reshape(x.shape[1], ROWS, LANES)


def _build_data(class_logits, box_regression, proposal_boxes):
    return jnp.concatenate(
        [_prep(class_logits), _prep(box_regression[:, 4:8]), _prep(proposal_boxes)],
        axis=0,
    )


def kernel(class_logits, box_regression, proposal_boxes):
    data = _build_data(class_logits, box_regression, proposal_boxes)
    out = pl.pallas_call(
        _nms_kernel,
        out_shape=jax.ShapeDtypeStruct((DETS, LANES), jnp.float32),
    )(data)
    return out[:, :5]
